# Initial kernel scaffold; baseline (speedup 1.0000x reference)
import jax
import jax.numpy as jnp
from jax.experimental import pallas as pl


def kernel(text_emb, mel_emb):
    raise NotImplementedError("write your pallas kernel here")


# speedup vs baseline: 10.8422x; 10.8422x over previous
import jax
import jax.numpy as jnp
from jax.experimental import pallas as pl
from jax.experimental.pallas import tpu as pltpu


def _bmm_kernel(mel_ref, text_ref, out_ref):
    lp = jax.lax.dot_general(
        mel_ref[0], text_ref[0], (((1,), (1,)), ((), ())),
        preferred_element_type=jnp.float32)
    out_ref[:, 0, 0, :] = -lp


def _dp_kernel(nlp_ref, out_ref, m_ref):
    n_mel, b, t = nlp_ref.shape
    inf = jnp.float32(jnp.inf)
    lane = jax.lax.broadcasted_iota(jnp.int32, (b, t), 1)
    infcol = jnp.full((b, 1), inf, jnp.float32)

    def move_mask(cur):
        sh = jnp.concatenate([infcol, cur[:, :-1]], axis=1)
        return jnp.where((sh <= cur) & (lane > 0), 1.0, 0.0).astype(jnp.float32)

    row0 = jnp.where(lane == 0, nlp_ref[0], inf)
    m_ref[0] = move_mask(row0)

    def fwd(i, prev):
        sh = jnp.concatenate([infcol, prev[:, :-1]], axis=1)
        cur = nlp_ref[i] + jnp.minimum(prev, sh)
        m_ref[i] = move_mask(cur)
        return cur

    jax.lax.fori_loop(1, n_mel, fwd, row0)

    path = jnp.where(lane == t - 1, 1.0, 0.0).astype(jnp.float32)
    out_ref[n_mel - 1] = path
    zcol = jnp.zeros((b, 1), jnp.float32)

    def bwd(s, p):
        k = n_mel - 2 - s
        mv = p * m_ref[k]
        p2 = p - mv + jnp.concatenate([mv[:, 1:], zcol], axis=1)
        out_ref[k] = p2
        return p2

    jax.lax.fori_loop(0, n_mel - 1, bwd, path)


def kernel(text_emb, mel_emb):
    b, n_mel, d = mel_emb.shape
    n_text = text_emb.shape[1]
    nlp_t = pl.pallas_call(
        _bmm_kernel,
        grid=(b,),
        in_specs=[
            pl.BlockSpec((1, n_mel, d), lambda i: (i, 0, 0)),
            pl.BlockSpec((1, n_text, d), lambda i: (i, 0, 0)),
        ],
        out_specs=pl.BlockSpec((n_mel, 1, 1, n_text), lambda i: (0, i, 0, 0)),
        out_shape=jax.ShapeDtypeStruct((n_mel, b, 1, n_text), jnp.float32),
    )(mel_emb, text_emb)
    nlp_t = nlp_t.reshape(n_mel, b, n_text)

    path_t = pl.pallas_call(
        _dp_kernel,
        in_specs=[pl.BlockSpec((n_mel, b, n_text), lambda: (0, 0, 0))],
        out_specs=pl.BlockSpec((n_mel, b, n_text), lambda: (0, 0, 0)),
        out_shape=jax.ShapeDtypeStruct((n_mel, b, n_text), jnp.float32),
        scratch_shapes=[pltpu.VMEM((n_mel, b, n_text), jnp.float32)],
    )(nlp_t)
    return jnp.transpose(path_t, (1, 0, 2))


# speedup vs baseline: 17.0573x; 1.5732x over previous
import jax
import jax.numpy as jnp
from jax.experimental import pallas as pl
from jax.experimental.pallas import tpu as pltpu


def _bmm_kernel(mel_ref, text_ref, out_ref):
    out_ref[0] = jax.lax.dot_general(
        mel_ref[0], text_ref[0], (((1,), (1,)), ((), ())),
        preferred_element_type=jnp.float32)


def _dp_kernel(lp_ref, out_ref, m_ref):
    b, n_mel, t = lp_ref.shape
    inf = jnp.float32(jnp.inf)
    lane = jax.lax.broadcasted_iota(jnp.int32, (b, t), 1)
    infcol = jnp.full((b, 1), inf, jnp.float32)

    row0 = jnp.where(lane == 0, -lp_ref[:, 0, :], inf)

    def fwd(i, prev):
        sh = jnp.concatenate([infcol, prev[:, :-1]], axis=1)
        m_ref[i - 1] = jnp.where((sh <= prev) & (lane > 0), 1.0, 0.0)
        cur = jnp.minimum(prev, sh) - lp_ref[:, i, :]
        return cur

    jax.lax.fori_loop(1, n_mel, fwd, row0)

    path = jnp.where(lane == t - 1, 1.0, 0.0).astype(jnp.float32)
    out_ref[:, n_mel - 1, :] = path
    zcol = jnp.zeros((b, 1), jnp.float32)

    def bwd(s, p):
        k = n_mel - 2 - s
        mv = p * m_ref[k]
        p2 = p - mv + jnp.concatenate([mv[:, 1:], zcol], axis=1)
        out_ref[:, k, :] = p2
        return p2

    jax.lax.fori_loop(0, n_mel - 1, bwd, path)


def kernel(text_emb, mel_emb):
    b, n_mel, d = mel_emb.shape
    n_text = text_emb.shape[1]
    lp = pl.pallas_call(
        _bmm_kernel,
        grid=(b,),
        in_specs=[
            pl.BlockSpec((1, n_mel, d), lambda i: (i, 0, 0)),
            pl.BlockSpec((1, n_text, d), lambda i: (i, 0, 0)),
        ],
        out_specs=pl.BlockSpec((1, n_mel, n_text), lambda i: (i, 0, 0)),
        out_shape=jax.ShapeDtypeStruct((b, n_mel, n_text), jnp.float32),
    )(mel_emb, text_emb)

    return pl.pallas_call(
        _dp_kernel,
        in_specs=[pl.BlockSpec((b, n_mel, n_text), lambda: (0, 0, 0))],
        out_specs=pl.BlockSpec((b, n_mel, n_text), lambda: (0, 0, 0)),
        out_shape=jax.ShapeDtypeStruct((b, n_mel, n_text), jnp.float32),
        scratch_shapes=[pltpu.VMEM((n_mel, b, n_text), jnp.float32)],
    )(lp)
